# baseline (speedup 1.0000x reference)
import functools

import jax
import jax.numpy as jnp
from jax import lax
from jax.experimental import pallas as pl
from jax.experimental.pallas import tpu as pltpu
from jax.experimental.pallas import tpu_sc as plsc

VOCAB = 1000000
EMBED = 64
HIDDEN = 256
OUT = 1024
B = 1024
T = 20


_NC, _NS = 2, 16
_NW = _NC * _NS
_N = B * T
_N_PER_W = _N // _NW
_C = 32
_NCH = _N_PER_W // _C
_TROW = 8


def _gather_body(idx_hbm, table_hbm, out_hbm, idx_v, tiles_v,
                 comp_v, sem_a, sem_b):
    wid = lax.axis_index("s") * _NC + lax.axis_index("c")
    pltpu.sync_copy(idx_hbm.at[pl.ds(wid * _N_PER_W, _N_PER_W)], idx_v)

    def issue(c, buf, sem):
        def issue_group(g, carry):
            idxv = idx_v[pl.ds(c * _C + g * 16, 16)]
            tidv = lax.shift_right_logical(idxv, 3)
            for j in range(16):
                pltpu.async_copy(
                    table_hbm.at[tidv[j]], tiles_v.at[buf, g * 16 + j], sem)
            return carry

        lax.fori_loop(0, _C // 16, issue_group, 0)

    def drain(buf, sem):
        pltpu.make_async_copy(
            table_hbm.at[pl.ds(0, _C)], tiles_v.at[buf], sem).wait()

    def extract(c, buf):
        def ext_group(g, carry):
            idxv = idx_v[pl.ds(c * _C + g * 16, 16)]
            subv = lax.rem(idxv, 8)
            for j in range(16):
                s = subv[j]
                r = g * 16 + j
                for k in range(EMBED // 16):
                    comp_v[r, pl.ds(k * 16, 16)] = (
                        tiles_v[buf, r, s, pl.ds(k * 16, 16)])
            return carry

        lax.fori_loop(0, _C // 16, ext_group, 0)
        pltpu.sync_copy(
            comp_v, out_hbm.at[pl.ds(wid * _N_PER_W + c * _C, _C)])

    issue(0, 0, sem_a)

    def pair_body(p, carry):
        c0 = p * 2
        drain(0, sem_a)
        issue(c0 + 1, 1, sem_b)
        extract(c0, 0)
        drain(1, sem_b)

        @pl.when(p + 1 < _NCH // 2)
        def _():
            issue(c0 + 2, 0, sem_a)

        extract(c0 + 1, 1)
        return carry

    lax.fori_loop(0, _NCH // 2, pair_body, 0)


_gather_call_cache = []


def _gather_call(idx, table3):
    if not _gather_call_cache:
        _gather_call_cache.append(functools.partial(
            pl.kernel,
            out_type=jax.ShapeDtypeStruct((_N, EMBED), jnp.float32),
            mesh=plsc.VectorSubcoreMesh(core_axis_name="c", subcore_axis_name="s"),
            scratch_types=[
                pltpu.VMEM((_N_PER_W,), jnp.int32),
                pltpu.VMEM((2, _C, _TROW, EMBED), jnp.float32),
                pltpu.VMEM((_C, EMBED), jnp.float32),
                pltpu.SemaphoreType.DMA,
                pltpu.SemaphoreType.DMA,
            ],
        )(_gather_body))
    return _gather_call_cache[0](idx, table3)


_NB = 1
_BB = B // _NB
_H4 = 4 * HIDDEN


def _sigmoid(v):
    return 1.0 / (1.0 + jnp.exp(-v))


def _lstm_body(e_ref, w0, b0, w1, b1, fcw, fcb,
               out_ref, hn_ref, cn_ref, h0, c0, h1, c1, xh0, xh1):
    h0[...] = jnp.zeros((_BB, HIDDEN), jnp.float32)
    c0[...] = jnp.zeros((_BB, HIDDEN), jnp.float32)
    h1[...] = jnp.zeros((_BB, HIDDEN), jnp.float32)
    c1[...] = jnp.zeros((_BB, HIDDEN), jnp.float32)
    xh0[...] = jnp.zeros((_BB, HIDDEN + 128), jnp.bfloat16)
    xh1[...] = jnp.zeros((_BB, 2 * HIDDEN), jnp.bfloat16)

    def gates(g, c_prev):
        i = _sigmoid(g[:, 0:HIDDEN])
        f = _sigmoid(g[:, HIDDEN:2 * HIDDEN])
        gg = jnp.tanh(g[:, 2 * HIDDEN:3 * HIDDEN])
        o = _sigmoid(g[:, 3 * HIDDEN:4 * HIDDEN])
        c_new = f * c_prev + i * gg
        h_new = o * jnp.tanh(c_new)
        return h_new, c_new

    def step(t, carry):
        xh0[:, HIDDEN:HIDDEN + 128] = e_ref[t]
        g0 = jnp.dot(xh0[...], w0[...],
                     preferred_element_type=jnp.float32) + b0[...]
        h0_new, c0_new = gates(g0, c0[...])
        h0[...] = h0_new
        c0[...] = c0_new
        h0_bf = h0_new.astype(jnp.bfloat16)
        xh0[:, 0:HIDDEN] = h0_bf
        xh1[:, 0:HIDDEN] = h0_bf
        g1 = jnp.dot(xh1[...], w1[...],
                     preferred_element_type=jnp.float32) + b1[...]
        h1_new, c1_new = gates(g1, c1[...])
        h1[...] = h1_new
        c1[...] = c1_new
        xh1[:, HIDDEN:2 * HIDDEN] = h1_new.astype(jnp.bfloat16)
        return carry

    lax.fori_loop(0, T, step, 0)

    out_ref[...] = (jnp.dot(h1[...], fcw[...], preferred_element_type=jnp.float32)
                    + fcb[...])
    hn_ref[0] = h0[...]
    hn_ref[1] = h1[...]
    cn_ref[0] = c0[...]
    cn_ref[1] = c1[...]


def _full(shape):
    return pl.BlockSpec(shape, lambda i: (0,) * len(shape))


_lstm_call = pl.pallas_call(
    _lstm_body,
    grid=(_NB,),
    in_specs=[
        pl.BlockSpec((T, _BB, 128), lambda i: (0, i, 0)),
        _full((HIDDEN + 128, _H4)),
        _full((1, _H4)),
        _full((2 * HIDDEN, _H4)),
        _full((1, _H4)),
        _full((HIDDEN, OUT)),
        _full((1, OUT)),
    ],
    out_specs=[
        pl.BlockSpec((_BB, OUT), lambda i: (i, 0)),
        pl.BlockSpec((2, _BB, HIDDEN), lambda i: (0, i, 0)),
        pl.BlockSpec((2, _BB, HIDDEN), lambda i: (0, i, 0)),
    ],
    out_shape=[
        jax.ShapeDtypeStruct((B, OUT), jnp.float32),
        jax.ShapeDtypeStruct((2, B, HIDDEN), jnp.float32),
        jax.ShapeDtypeStruct((2, B, HIDDEN), jnp.float32),
    ],
    scratch_shapes=[pltpu.VMEM((_BB, HIDDEN), jnp.float32)] * 4 + [
        pltpu.VMEM((_BB, HIDDEN + 128), jnp.bfloat16),
        pltpu.VMEM((_BB, 2 * HIDDEN), jnp.bfloat16),
    ],
    compiler_params=pltpu.CompilerParams(
        dimension_semantics=("arbitrary",),
    ),
)


def kernel(x, emb, W_ih_l0, W_hh_l0, b_ih_l0, b_hh_l0,
           W_ih_l1, W_hh_l1, b_ih_l1, b_hh_l1, fc_W, fc_b):
    idx = x.T.reshape(_N)
    table3 = emb.reshape(VOCAB // _TROW, _TROW, EMBED)
    e_flat = _gather_call(idx, table3)
    e2 = jnp.pad(e_flat.astype(jnp.bfloat16).reshape(T, B, EMBED),
                 ((0, 0), (0, 0), (0, 128 - EMBED)))

    w0 = jnp.concatenate(
        [W_hh_l0.T, W_ih_l0.T,
         jnp.zeros((128 - EMBED, _H4), jnp.float32)], axis=0
    ).astype(jnp.bfloat16)
    w1 = jnp.concatenate([W_ih_l1.T, W_hh_l1.T], axis=0).astype(jnp.bfloat16)

    out, h_n, c_n = _lstm_call(
        e2,
        w0, (b_ih_l0 + b_hh_l0).reshape(1, _H4),
        w1, (b_ih_l1 + b_hh_l1).reshape(1, _H4),
        fc_W.T, fc_b.reshape(1, OUT),
    )
    return (out, h_n, c_n)


# speedup vs baseline: 1.0173x; 1.0173x over previous
import functools

import jax
import jax.numpy as jnp
from jax import lax
from jax.experimental import pallas as pl
from jax.experimental.pallas import tpu as pltpu
from jax.experimental.pallas import tpu_sc as plsc

VOCAB = 1000000
EMBED = 64
HIDDEN = 256
OUT = 1024
B = 1024
T = 20


_NC, _NS = 2, 16
_NW = _NC * _NS
_N = B * T
_N_PER_W = _N // _NW
_C = 32
_NCH = _N_PER_W // _C
_TROW = 8


def _gather_body(idx_hbm, table_hbm, out_hbm, idx_v, tiles_v,
                 comp_v, sem_a, sem_b):
    wid = lax.axis_index("s") * _NC + lax.axis_index("c")
    pltpu.sync_copy(idx_hbm.at[pl.ds(wid * _N_PER_W, _N_PER_W)], idx_v)

    def issue(c, buf, sem):
        def issue_group(g, carry):
            idxv = idx_v[pl.ds(c * _C + g * 16, 16)]
            tidv = lax.shift_right_logical(idxv, 3)
            for j in range(16):
                pltpu.async_copy(
                    table_hbm.at[tidv[j]], tiles_v.at[buf, g * 16 + j], sem)
            return carry

        lax.fori_loop(0, _C // 16, issue_group, 0)

    def drain(buf, sem):
        pltpu.make_async_copy(
            table_hbm.at[pl.ds(0, _C)], tiles_v.at[buf], sem).wait()

    def extract(c, buf):
        def ext_group(g, carry):
            idxv = idx_v[pl.ds(c * _C + g * 16, 16)]
            subv = lax.rem(idxv, 8)
            for j in range(16):
                s = subv[j]
                r = g * 16 + j
                for k in range(EMBED // 16):
                    comp_v[r, pl.ds(k * 16, 16)] = (
                        tiles_v[buf, r, s, pl.ds(k * 16, 16)])
            return carry

        lax.fori_loop(0, _C // 16, ext_group, 0)
        pltpu.sync_copy(
            comp_v, out_hbm.at[pl.ds(wid * _N_PER_W + c * _C, _C)])

    issue(0, 0, sem_a)

    def pair_body(p, carry):
        c0 = p * 2
        drain(0, sem_a)
        issue(c0 + 1, 1, sem_b)
        extract(c0, 0)
        drain(1, sem_b)

        @pl.when(p + 1 < _NCH // 2)
        def _():
            issue(c0 + 2, 0, sem_a)

        extract(c0 + 1, 1)
        return carry

    lax.fori_loop(0, _NCH // 2, pair_body, 0)


_gather_call_cache = []


def _gather_call(idx, table3):
    if not _gather_call_cache:
        _gather_call_cache.append(functools.partial(
            pl.kernel,
            out_type=jax.ShapeDtypeStruct((_N, EMBED), jnp.float32),
            mesh=plsc.VectorSubcoreMesh(core_axis_name="c", subcore_axis_name="s"),
            scratch_types=[
                pltpu.VMEM((_N_PER_W,), jnp.int32),
                pltpu.VMEM((2, _C, _TROW, EMBED), jnp.float32),
                pltpu.VMEM((_C, EMBED), jnp.float32),
                pltpu.SemaphoreType.DMA,
                pltpu.SemaphoreType.DMA,
            ],
        )(_gather_body))
    return _gather_call_cache[0](idx, table3)


_NB = 1
_BB = B // _NB
_H4 = 4 * HIDDEN


def _sigmoid(v):
    return 0.5 * jnp.tanh(0.5 * v) + 0.5


def _lstm_body(e_ref, w0, b0, w1, b1, fcw, fcb,
               out_ref, hn_ref, cn_ref, h0, c0, h1, c1, xh0, xh1):
    h0[...] = jnp.zeros((_BB, HIDDEN), jnp.float32)
    c0[...] = jnp.zeros((_BB, HIDDEN), jnp.float32)
    h1[...] = jnp.zeros((_BB, HIDDEN), jnp.float32)
    c1[...] = jnp.zeros((_BB, HIDDEN), jnp.float32)
    xh0[...] = jnp.zeros((_BB, HIDDEN + 128), jnp.bfloat16)
    xh1[...] = jnp.zeros((_BB, 2 * HIDDEN), jnp.bfloat16)

    def gates(g, c_prev):
        i = _sigmoid(g[:, 0:HIDDEN])
        f = _sigmoid(g[:, HIDDEN:2 * HIDDEN])
        gg = jnp.tanh(g[:, 2 * HIDDEN:3 * HIDDEN])
        o = _sigmoid(g[:, 3 * HIDDEN:4 * HIDDEN])
        c_new = f * c_prev + i * gg
        h_new = o * jnp.tanh(c_new)
        return h_new, c_new

    def step(t, carry):
        xh0[:, HIDDEN:HIDDEN + 128] = e_ref[t]
        g0 = jnp.dot(xh0[...], w0[...],
                     preferred_element_type=jnp.float32) + b0[...]
        h0_new, c0_new = gates(g0, c0[...])
        h0[...] = h0_new
        c0[...] = c0_new
        h0_bf = h0_new.astype(jnp.bfloat16)
        xh0[:, 0:HIDDEN] = h0_bf
        xh1[:, 0:HIDDEN] = h0_bf
        g1 = jnp.dot(xh1[...], w1[...],
                     preferred_element_type=jnp.float32) + b1[...]
        h1_new, c1_new = gates(g1, c1[...])
        h1[...] = h1_new
        c1[...] = c1_new
        xh1[:, HIDDEN:2 * HIDDEN] = h1_new.astype(jnp.bfloat16)
        return carry

    lax.fori_loop(0, T, step, 0)

    out_ref[...] = (jnp.dot(h1[...], fcw[...], preferred_element_type=jnp.float32)
                    + fcb[...])
    hn_ref[0] = h0[...]
    hn_ref[1] = h1[...]
    cn_ref[0] = c0[...]
    cn_ref[1] = c1[...]


def _full(shape):
    return pl.BlockSpec(shape, lambda i: (0,) * len(shape))


_lstm_call = pl.pallas_call(
    _lstm_body,
    grid=(_NB,),
    in_specs=[
        pl.BlockSpec((T, _BB, 128), lambda i: (0, i, 0)),
        _full((HIDDEN + 128, _H4)),
        _full((1, _H4)),
        _full((2 * HIDDEN, _H4)),
        _full((1, _H4)),
        _full((HIDDEN, OUT)),
        _full((1, OUT)),
    ],
    out_specs=[
        pl.BlockSpec((_BB, OUT), lambda i: (i, 0)),
        pl.BlockSpec((2, _BB, HIDDEN), lambda i: (0, i, 0)),
        pl.BlockSpec((2, _BB, HIDDEN), lambda i: (0, i, 0)),
    ],
    out_shape=[
        jax.ShapeDtypeStruct((B, OUT), jnp.float32),
        jax.ShapeDtypeStruct((2, B, HIDDEN), jnp.float32),
        jax.ShapeDtypeStruct((2, B, HIDDEN), jnp.float32),
    ],
    scratch_shapes=[pltpu.VMEM((_BB, HIDDEN), jnp.float32)] * 4 + [
        pltpu.VMEM((_BB, HIDDEN + 128), jnp.bfloat16),
        pltpu.VMEM((_BB, 2 * HIDDEN), jnp.bfloat16),
    ],
    compiler_params=pltpu.CompilerParams(
        dimension_semantics=("arbitrary",),
    ),
)


def kernel(x, emb, W_ih_l0, W_hh_l0, b_ih_l0, b_hh_l0,
           W_ih_l1, W_hh_l1, b_ih_l1, b_hh_l1, fc_W, fc_b):
    idx = x.T.reshape(_N)
    table3 = emb.reshape(VOCAB // _TROW, _TROW, EMBED)
    e_flat = _gather_call(idx, table3)
    e2 = jnp.pad(e_flat.astype(jnp.bfloat16).reshape(T, B, EMBED),
                 ((0, 0), (0, 0), (0, 128 - EMBED)))

    w0 = jnp.concatenate(
        [W_hh_l0.T, W_ih_l0.T,
         jnp.zeros((128 - EMBED, _H4), jnp.float32)], axis=0
    ).astype(jnp.bfloat16)
    w1 = jnp.concatenate([W_ih_l1.T, W_hh_l1.T], axis=0).astype(jnp.bfloat16)

    out, h_n, c_n = _lstm_call(
        e2,
        w0, (b_ih_l0 + b_hh_l0).reshape(1, _H4),
        w1, (b_ih_l1 + b_hh_l1).reshape(1, _H4),
        fc_W.T, fc_b.reshape(1, OUT),
    )
    return (out, h_n, c_n)
